# baseline (speedup 1.0000x reference)
import functools

import jax
import jax.numpy as jnp
from jax import lax
from jax.experimental import pallas as pl
from jax.experimental.pallas import tpu as pltpu
from jax.experimental.pallas import tpu_sc as plsc

E = 320000
T = 1280000
EMB = 128
INT_EMB = 64

F32 = jnp.float32
I32 = jnp.int32


def _silu(x):
    return x * (1.0 / (1.0 + jnp.exp(-x)))


def _bdot(a, w):
    return jnp.dot(a.astype(jnp.bfloat16), w.astype(jnp.bfloat16),
                   preferred_element_type=F32)


def _ileave_bf16(h):
    blk = h.shape[0]
    parts = []
    for g in range(2):
        a = h[:, g * 32:g * 32 + 16]
        b = h[:, g * 32 + 16:g * 32 + 32]
        parts.append(jnp.stack([a, b], axis=-1).reshape(blk, 32))
    return jnp.concatenate(parts, axis=1).astype(jnp.bfloat16)


def _prologue_body(x0_ref, rbf_ref, wkj_ref, bkj_ref, wr1_ref, wr2_ref,
                   wdn_ref, xdown_ref):
    x0 = x0_ref[...]
    h = _silu(_bdot(x0, wkj_ref[...]) + bkj_ref[...])
    rbf_e = jnp.dot(jnp.dot(rbf_ref[...], wr1_ref[...],
                            preferred_element_type=F32),
                    wr2_ref[...], preferred_element_type=F32)
    h = h * rbf_e
    xdown_ref[...] = _ileave_bf16(_silu(_bdot(h, wdn_ref[...])))


def _prologue(x0, rbf, w_kj, b_kj, w_rbf1, w_rbf2, w_down):
    blk = 2000
    grid = E // blk
    full = lambda a: pl.BlockSpec(a.shape, lambda i: (0,) * a.ndim)
    b_kj2 = b_kj.reshape(1, EMB)
    return pl.pallas_call(
        _prologue_body,
        grid=(grid,),
        in_specs=[
            pl.BlockSpec((blk, EMB), lambda i: (i, 0)),
            pl.BlockSpec((blk, rbf.shape[1]), lambda i: (i, 0)),
            full(w_kj), full(b_kj2), full(w_rbf1), full(w_rbf2), full(w_down),
        ],
        out_specs=pl.BlockSpec((blk, INT_EMB), lambda i: (i, 0)),
        out_shape=jax.ShapeDtypeStruct((E, INT_EMB), jnp.bfloat16),
    )(x0, rbf, w_kj, b_kj2, w_rbf1, w_rbf2, w_down)


def _sbf_body(sbf_ref, w1_ref, w2_ref, out_ref):
    out_ref[...] = _ileave_bf16(jnp.dot(
        jnp.dot(sbf_ref[...], w1_ref[...], preferred_element_type=F32),
        w2_ref[...], preferred_element_type=F32))


def _sbf_transform(sbf, w_sbf1, w_sbf2):
    blk = 2000
    grid = T // blk
    full = lambda a: pl.BlockSpec(a.shape, lambda i: (0,) * a.ndim)
    return pl.pallas_call(
        _sbf_body,
        grid=(grid,),
        in_specs=[
            pl.BlockSpec((blk, sbf.shape[1]), lambda i: (i, 0)),
            full(w_sbf1), full(w_sbf2),
        ],
        out_specs=pl.BlockSpec((blk, INT_EMB), lambda i: (i, 0)),
        out_shape=jax.ShapeDtypeStruct((T, INT_EMB), jnp.bfloat16),
    )(sbf, w_sbf1, w_sbf2)


def _make_sc_segment(e_total, t_total, n_chunks, blk_ids, gfix,
                     interpret=False):
    info_tiles = 16
    n_sc = 2
    ec = e_total // n_chunks
    npc = n_chunks // n_sc
    t_per_tile = t_total // info_tiles
    nblk = t_per_tile // blk_ids
    ngrp = blk_ids // 16
    d = INT_EMB
    rows_per_tile = ec // info_tiles
    zrows = next(z for z in (200, 128, 125, 100, 80, 50, 25)
                 if rows_per_tile % z == 0 and z <= gfix)
    nzcp = rows_per_tile // zrows
    nring = 16
    gsh = gfix.bit_length() - 1
    assert (1 << gsh) == gfix
    assert ec * n_chunks == e_total and t_per_tile * info_tiles == t_total
    assert nblk * blk_ids == t_per_tile and ngrp * 16 == blk_ids
    assert nzcp * zrows * info_tiles == ec and ngrp % 2 == 0
    assert nring * gfix >= blk_ids + 2 * gfix
    assert zrows <= gfix

    mesh = plsc.VectorSubcoreMesh(core_axis_name="c", subcore_axis_name="s",
                                  num_cores=n_sc, num_subcores=info_tiles)

    @functools.partial(
        pl.kernel,
        out_type=jax.ShapeDtypeStruct((e_total, d), F32),
        mesh=mesh,
        scratch_types=[
            pltpu.VMEM((2, blk_ids), I32),
            pltpu.VMEM((2, blk_ids), I32),
            pltpu.VMEM((nring, gfix), I32),
            pltpu.VMEM((nring, gfix), I32),
            pltpu.VMEM((nring, gfix), I32),
            pltpu.VMEM((gfix, d), jnp.bfloat16),
            pltpu.VMEM((gfix, d), jnp.bfloat16),
            pltpu.VMEM((gfix, d), F32),
            pltpu.VMEM_SHARED((ec + 16, d), F32),
            pltpu.SemaphoreType.DMA,
            pltpu.SemaphoreType.DMA,
            pltpu.SemaphoreType.DMA,
            pltpu.SemaphoreType.DMA,
        ],
        compiler_params=pltpu.CompilerParams(needs_layout_passes=False,
                                             use_tc_tiling_on_sc=False),
        interpret=interpret,
    )
    def sc_segment(xdown_hbm, sbfe_hbm, idr_hbm, ide_hbm, out_hbm,
                   idr_v, ide_v, dstb, srcb, ttb, xrows, srows, prod,
                   accum, sem1, sem2, sem3, sem4):
        sc_id = lax.axis_index("c")
        s = lax.axis_index("s")
        lane = lax.iota(I32, 16)
        nr1 = nring - 1

        for j in range(nring):
            def _init(g, _):
                v = (j * gfix + g * 16) + lane
                srcb[j, pl.ds(g * 16, 16)] = v
                ttb[j, pl.ds(g * 16, 16)] = v
                return 0
            lax.fori_loop(0, gfix // 16, _init, 0)

        def _zero_prod():
            def _zrow(r, _):
                for cg in range(d // 16):
                    prod[r, pl.ds(cg * 16, 16)] = jnp.zeros((16,), F32)
                return 0
            lax.fori_loop(0, zrows, _zrow, 0)

        def start_fire(j):
            pltpu.make_async_copy(xdown_hbm.at[srcb.at[j]], xrows, sem1).start()
            pltpu.make_async_copy(sbfe_hbm.at[ttb.at[j]], srows, sem2).start()

        def finish_fire(j):
            pltpu.make_async_copy(xdown_hbm.at[srcb.at[j]], xrows, sem1).wait()
            pltpu.make_async_copy(sbfe_hbm.at[ttb.at[j]], srows, sem2).wait()

            def _mul(r, _):
                for u in range(4):
                    row = r * 4 + u
                    for g2 in range(d // 32):
                        sl32 = pl.ds(g2 * 32, 32)
                        xa, xb = plsc.unpack(
                            xrows[row, sl32],
                            format=plsc.PackFormat.INTERLEAVED)
                        sa, sb = plsc.unpack(
                            srows[row, sl32],
                            format=plsc.PackFormat.INTERLEAVED)
                        prod[row, pl.ds(g2 * 32, 16)] = xa * sa
                        prod[row, pl.ds(g2 * 32 + 16, 16)] = xb * sb
                return 0
            lax.fori_loop(0, gfix // 4, _mul, 0)
            pltpu.sync_copy(prod, accum.at[dstb.at[j]], add=True)

        def pump(cons, pend, off_s):
            def cond(st):
                return off_s - st[0] >= gfix

            def body(st):
                cons_, pend_ = st

                @pl.when(pend_ == 1)
                def _():
                    finish_fire(((cons_ >> gsh) + nr1) & nr1)
                start_fire((cons_ >> gsh) & nr1)
                return (cons_ + gfix, jnp.int32(1))

            return lax.while_loop(cond, body, (cons, pend))

        def chunk_body(c, _):
            c0 = (sc_id * npc + c) * ec
            row0 = s * rows_per_tile
            base0 = s * t_per_tile
            _zero_prod()
            for k in range(nzcp):
                pltpu.sync_copy(prod.at[pl.ds(0, zrows)],
                                accum.at[pl.ds(row0 + k * zrows, zrows)])
            plsc.subcore_barrier()
            pltpu.sync_copy(idr_hbm.at[pl.ds(base0, blk_ids)], idr_v.at[0])
            pltpu.sync_copy(ide_hbm.at[pl.ds(base0, blk_ids)], ide_v.at[0])

            def blk_body(b, carry):
                off_vec, cons, pend = carry
                par = b & 1
                nb = b + 1
                nbase = base0 + nb * blk_ids
                npar = nb & 1

                @pl.when(nb < nblk)
                def _():
                    pltpu.make_async_copy(
                        idr_hbm.at[pl.ds(nbase, blk_ids)], idr_v.at[npar],
                        sem3).start()
                    pltpu.make_async_copy(
                        ide_hbm.at[pl.ds(nbase, blk_ids)], ide_v.at[npar],
                        sem4).start()

                base = base0 + b * blk_ids

                def grp_pair(ip, off_vec):
                    for u in range(2):
                        i = ip * 2 + u
                        sl = pl.ds(i * 16, 16)
                        loc = idr_v[par, sl] - c0
                        m = loc.astype(jnp.uint32) < jnp.uint32(ec)
                        cnt_vec = plsc.all_reduce_population_count(m)

                        @pl.when(jnp.any(m))
                        def _(loc=loc, m=m, i=i, off_vec=off_vec, sl=sl):
                            mi = m.astype(I32)
                            pos = off_vec + plsc.cumsum(mi) - 1
                            slc = (pos >> gsh) & nr1
                            col = pos & (gfix - 1)
                            plsc.store_scatter(dstb, [slc, col], loc, mask=m)
                            plsc.store_scatter(srcb, [slc, col],
                                               ide_v[par, sl], mask=m)
                            plsc.store_scatter(ttb, [slc, col],
                                               base + i * 16 + lane, mask=m)
                        off_vec = off_vec + cnt_vec
                    return off_vec

                off_vec = lax.fori_loop(0, ngrp // 2, grp_pair, off_vec)
                off_s = jnp.max(off_vec)
                cons, pend = pump(cons, pend, off_s)

                @pl.when(nb < nblk)
                def _():
                    pltpu.make_async_copy(
                        idr_hbm.at[pl.ds(nbase, blk_ids)], idr_v.at[npar],
                        sem3).wait()
                    pltpu.make_async_copy(
                        ide_hbm.at[pl.ds(nbase, blk_ids)], ide_v.at[npar],
                        sem4).wait()
                return (off_vec, cons, pend)

            carry0 = (jnp.zeros((16,), I32), jnp.int32(0), jnp.int32(0))
            off_vec, cons, pend = lax.fori_loop(0, nblk, blk_body, carry0)

            @pl.when(pend == 1)
            def _():
                finish_fire(((cons >> gsh) + nr1) & nr1)
            off_s = jnp.max(off_vec)
            rem = off_s - cons
            jd = (cons >> gsh) & nr1
            jd_vec = jnp.zeros((16,), I32) + jd
            for g in range(gfix // 16):
                p = g * 16 + lane
                plsc.store_scatter(dstb, [jd_vec, p], ec + lane,
                                   mask=(p >= rem))
            start_fire(jd)
            finish_fire(jd)
            plsc.subcore_barrier()
            for k in range(nzcp):
                rsl = pl.ds(row0 + k * zrows, zrows)
                pltpu.sync_copy(accum.at[rsl],
                                out_hbm.at[pl.ds(c0 + row0 + k * zrows, zrows)])
            plsc.subcore_barrier()
            return 0

        lax.fori_loop(0, npc, chunk_body, 0)

    return sc_segment


def _epilogue_body(x0_ref, seg_ref, wji_ref, bji_ref, wup_ref,
                   wb1a_ref, bb1a_ref, wb1b_ref, bb1b_ref,
                   wfbs_ref, bfbs_ref,
                   wa1a_ref, ba1a_ref, wa1b_ref, ba1b_ref,
                   wa2a_ref, ba2a_ref, wa2b_ref, ba2b_ref, out_ref):
    x0 = x0_ref[...]

    def mm(a, w):
        return _bdot(a, w[...])

    def res(x, wa, ba, wb, bb):
        h = _silu(mm(x, wa) + ba[...])
        h = _silu(mm(h, wb) + bb[...])
        return x + h

    x_ji = _silu(mm(x0, wji_ref) + bji_ref[...])
    x_kj = _silu(mm(seg_ref[...], wup_ref))
    x2 = x_ji + x_kj
    x2 = res(x2, wb1a_ref, bb1a_ref, wb1b_ref, bb1b_ref)
    x2 = _silu(mm(x2, wfbs_ref) + bfbs_ref[...])
    x = x0 + x2
    x = res(x, wa1a_ref, ba1a_ref, wa1b_ref, ba1b_ref)
    x = res(x, wa2a_ref, ba2a_ref, wa2b_ref, ba2b_ref)
    out_ref[...] = x


def _epilogue(x0, seg, w_ji, b_ji, w_up, w_bs1a, b_bs1a, w_bs1b, b_bs1b,
              w_fbs, b_fbs, w_as1a, b_as1a, w_as1b, b_as1b,
              w_as2a, b_as2a, w_as2b, b_as2b):
    blk = 2000
    grid = E // blk
    full = lambda a: pl.BlockSpec(a.shape, lambda i: (0,) * a.ndim)
    args = [w_ji, b_ji.reshape(1, EMB), w_up,
            w_bs1a, b_bs1a.reshape(1, EMB), w_bs1b, b_bs1b.reshape(1, EMB),
            w_fbs, b_fbs.reshape(1, EMB),
            w_as1a, b_as1a.reshape(1, EMB), w_as1b, b_as1b.reshape(1, EMB),
            w_as2a, b_as2a.reshape(1, EMB), w_as2b, b_as2b.reshape(1, EMB)]
    return pl.pallas_call(
        _epilogue_body,
        grid=(grid,),
        in_specs=[
            pl.BlockSpec((blk, EMB), lambda i: (i, 0)),
            pl.BlockSpec((blk, INT_EMB), lambda i: (i, 0)),
        ] + [full(a) for a in args],
        out_specs=pl.BlockSpec((blk, EMB), lambda i: (i, 0)),
        out_shape=jax.ShapeDtypeStruct((E, EMB), F32),
    )(x0, seg, *args)


def kernel(x0, rbf, sbf, id_expand_kj, id_reduce_ji, R,
           W_rbf1, W_rbf2, W_sbf1, W_sbf2, W_ji, b_ji, W_kj, b_kj,
           W_down, W_up, W_bs1a, b_bs1a, W_bs1b, b_bs1b, W_fbs, b_fbs,
           W_as1a, b_as1a, W_as1b, b_as1b, W_as2a, b_as2a, W_as2b, b_as2b):
    x_down = _prologue(x0, rbf, W_kj, b_kj, W_rbf1, W_rbf2, W_down)
    sbf_e = _sbf_transform(sbf, W_sbf1, W_sbf2)
    sc_seg = _make_sc_segment(E, T, n_chunks=16, blk_ids=1600, gfix=128)
    seg = sc_seg(x_down, sbf_e, id_reduce_ji, id_expand_kj)
    return _epilogue(x0, seg, W_ji, b_ji, W_up, W_bs1a, b_bs1a,
                     W_bs1b, b_bs1b, W_fbs, b_fbs, W_as1a, b_as1a,
                     W_as1b, b_as1b, W_as2a, b_as2a, W_as2b, b_as2b)


# speedup vs baseline: 4.3933x; 4.3933x over previous
import functools

import jax
import jax.numpy as jnp
from jax import lax
from jax.experimental import pallas as pl
from jax.experimental.pallas import tpu as pltpu
from jax.experimental.pallas import tpu_sc as plsc

E = 320000
T = 1280000
EMB = 128
INT_EMB = 64

F32 = jnp.float32
I32 = jnp.int32


def _silu(x):
    return x * (1.0 / (1.0 + jnp.exp(-x)))


def _bdot(a, w):
    return jnp.dot(a.astype(jnp.bfloat16), w.astype(jnp.bfloat16),
                   preferred_element_type=F32)


def _seg_perm():
    perm = []
    for g in range(INT_EMB // 32):
        perm += [32 * g + 2 * i for i in range(16)]
        perm += [32 * g + 2 * i + 1 for i in range(16)]
    return jnp.asarray(perm, dtype=jnp.int32)


def _prologue_body(x0_ref, rbf_ref, wkj_ref, bkj_ref, wr1_ref, wr2_ref,
                   wdn_ref, xdown_ref):
    x0 = x0_ref[...]
    h = _silu(_bdot(x0, wkj_ref[...]) + bkj_ref[...])
    rbf_e = jnp.dot(jnp.dot(rbf_ref[...], wr1_ref[...],
                            preferred_element_type=F32),
                    wr2_ref[...], preferred_element_type=F32)
    h = h * rbf_e
    xdown_ref[...] = _silu(_bdot(h, wdn_ref[...])).astype(jnp.bfloat16)


def _prologue(x0, rbf, w_kj, b_kj, w_rbf1, w_rbf2, w_down):
    blk = 2000
    grid = E // blk
    full = lambda a: pl.BlockSpec(a.shape, lambda i: (0,) * a.ndim)
    b_kj2 = b_kj.reshape(1, EMB)
    return pl.pallas_call(
        _prologue_body,
        grid=(grid,),
        in_specs=[
            pl.BlockSpec((blk, EMB), lambda i: (i, 0)),
            pl.BlockSpec((blk, rbf.shape[1]), lambda i: (i, 0)),
            full(w_kj), full(b_kj2), full(w_rbf1), full(w_rbf2), full(w_down),
        ],
        out_specs=pl.BlockSpec((blk, INT_EMB), lambda i: (i, 0)),
        out_shape=jax.ShapeDtypeStruct((E, INT_EMB), jnp.bfloat16),
    )(x0, rbf, w_kj, b_kj2, w_rbf1, w_rbf2, w_down)


def _sbf_body(sbf_ref, w1_ref, w2_ref, out_ref):
    out_ref[...] = jnp.dot(
        jnp.dot(sbf_ref[...], w1_ref[...], preferred_element_type=F32),
        w2_ref[...], preferred_element_type=F32).astype(jnp.bfloat16)


def _sbf_transform(sbf, w_sbf1, w_sbf2):
    blk = 2000
    grid = T // blk
    full = lambda a: pl.BlockSpec(a.shape, lambda i: (0,) * a.ndim)
    return pl.pallas_call(
        _sbf_body,
        grid=(grid,),
        in_specs=[
            pl.BlockSpec((blk, sbf.shape[1]), lambda i: (i, 0)),
            full(w_sbf1), full(w_sbf2),
        ],
        out_specs=pl.BlockSpec((blk, INT_EMB), lambda i: (i, 0)),
        out_shape=jax.ShapeDtypeStruct((T, INT_EMB), jnp.bfloat16),
    )(sbf, w_sbf1, w_sbf2)


def _make_sc_segment(e_total, t_total, n_chunks, blk_ids, gfix,
                     interpret=False):
    info_tiles = 16
    n_sc = 2
    ec = e_total // n_chunks
    npc = n_chunks // n_sc
    t_per_tile = t_total // info_tiles
    nblk = t_per_tile // blk_ids
    ngrp = blk_ids // 16
    d = INT_EMB
    rows_per_tile = ec // info_tiles
    zrows = next(z for z in (200, 128, 125, 100, 80, 50, 25)
                 if rows_per_tile % z == 0 and z <= gfix)
    nzcp = rows_per_tile // zrows
    nring = 16
    gsh = gfix.bit_length() - 1
    assert (1 << gsh) == gfix
    assert ec * n_chunks == e_total and t_per_tile * info_tiles == t_total
    assert nblk * blk_ids == t_per_tile and ngrp * 16 == blk_ids
    assert nzcp * zrows * info_tiles == ec and ngrp % 2 == 0
    assert nring * gfix >= blk_ids + 2 * gfix
    assert zrows <= gfix

    mesh = plsc.VectorSubcoreMesh(core_axis_name="c", subcore_axis_name="s",
                                  num_cores=n_sc, num_subcores=info_tiles)

    @functools.partial(
        pl.kernel,
        out_type=jax.ShapeDtypeStruct((e_total, d), F32),
        mesh=mesh,
        scratch_types=[
            pltpu.VMEM((2, blk_ids), I32),
            pltpu.VMEM((2, blk_ids), I32),
            pltpu.VMEM((nring, gfix), I32),
            pltpu.VMEM((nring, gfix), I32),
            pltpu.VMEM((nring, gfix), I32),
            pltpu.VMEM((gfix, d), jnp.bfloat16),
            pltpu.VMEM((gfix, d), jnp.bfloat16),
            pltpu.VMEM((gfix, d), F32),
            pltpu.VMEM_SHARED((ec + 16, d), F32),
            pltpu.SemaphoreType.DMA,
            pltpu.SemaphoreType.DMA,
            pltpu.SemaphoreType.DMA,
            pltpu.SemaphoreType.DMA,
        ],
        compiler_params=pltpu.CompilerParams(needs_layout_passes=False,
                                             use_tc_tiling_on_sc=False),
        interpret=interpret,
    )
    def sc_segment(xdown_hbm, sbfe_hbm, idr_hbm, ide_hbm, out_hbm,
                   idr_v, ide_v, dstb, srcb, ttb, xrows, srows, prod,
                   accum, sem1, sem2, sem3, sem4):
        sc_id = lax.axis_index("c")
        s = lax.axis_index("s")
        lane = lax.iota(I32, 16)
        nr1 = nring - 1

        for j in range(nring):
            def _init(g, _):
                v = (j * gfix + g * 16) + lane
                srcb[j, pl.ds(g * 16, 16)] = v
                ttb[j, pl.ds(g * 16, 16)] = v
                return 0
            lax.fori_loop(0, gfix // 16, _init, 0)

        def _zero_prod():
            def _zrow(r, _):
                for cg in range(d // 16):
                    prod[r, pl.ds(cg * 16, 16)] = jnp.zeros((16,), F32)
                return 0
            lax.fori_loop(0, zrows, _zrow, 0)

        def start_fire(j):
            pltpu.make_async_copy(xdown_hbm.at[srcb.at[j]], xrows, sem1).start()
            pltpu.make_async_copy(sbfe_hbm.at[ttb.at[j]], srows, sem2).start()

        def finish_fire(j):
            pltpu.make_async_copy(xdown_hbm.at[srcb.at[j]], xrows, sem1).wait()
            pltpu.make_async_copy(sbfe_hbm.at[ttb.at[j]], srows, sem2).wait()

            def _mul(r, _):
                for u in range(4):
                    row = r * 4 + u
                    for g2 in range(d // 32):
                        sl32 = pl.ds(g2 * 32, 32)
                        xa, xb = plsc.unpack(
                            xrows[row, sl32],
                            format=plsc.PackFormat.INTERLEAVED)
                        sa, sb = plsc.unpack(
                            srows[row, sl32],
                            format=plsc.PackFormat.INTERLEAVED)
                        prod[row, pl.ds(g2 * 32, 16)] = xa * sa
                        prod[row, pl.ds(g2 * 32 + 16, 16)] = xb * sb
                return 0
            lax.fori_loop(0, gfix // 4, _mul, 0)
            pltpu.sync_copy(prod, accum.at[dstb.at[j]], add=True)

        def pump(cons, pend, off_s):
            def cond(st):
                return off_s - st[0] >= gfix

            def body(st):
                cons_, pend_ = st

                @pl.when(pend_ == 1)
                def _():
                    finish_fire(((cons_ >> gsh) + nr1) & nr1)
                start_fire((cons_ >> gsh) & nr1)
                return (cons_ + gfix, jnp.int32(1))

            return lax.while_loop(cond, body, (cons, pend))

        def chunk_body(c, _):
            c0 = (sc_id * npc + c) * ec
            row0 = s * rows_per_tile
            base0 = s * t_per_tile
            _zero_prod()
            for k in range(nzcp):
                pltpu.sync_copy(prod.at[pl.ds(0, zrows)],
                                accum.at[pl.ds(row0 + k * zrows, zrows)])
            plsc.subcore_barrier()
            pltpu.sync_copy(idr_hbm.at[pl.ds(base0, blk_ids)], idr_v.at[0])
            pltpu.sync_copy(ide_hbm.at[pl.ds(base0, blk_ids)], ide_v.at[0])

            def blk_body(b, carry):
                off_vec, cons, pend = carry
                par = b & 1
                nb = b + 1
                nbase = base0 + nb * blk_ids
                npar = nb & 1

                @pl.when(nb < nblk)
                def _():
                    pltpu.make_async_copy(
                        idr_hbm.at[pl.ds(nbase, blk_ids)], idr_v.at[npar],
                        sem3).start()
                    pltpu.make_async_copy(
                        ide_hbm.at[pl.ds(nbase, blk_ids)], ide_v.at[npar],
                        sem4).start()

                base = base0 + b * blk_ids

                def grp_pair(ip, off_vec):
                    for u in range(2):
                        i = ip * 2 + u
                        sl = pl.ds(i * 16, 16)
                        loc = idr_v[par, sl] - c0
                        m = loc.astype(jnp.uint32) < jnp.uint32(ec)
                        cnt_vec = plsc.all_reduce_population_count(m)

                        @pl.when(jnp.any(m))
                        def _(loc=loc, m=m, i=i, off_vec=off_vec, sl=sl):
                            mi = m.astype(I32)
                            pos = off_vec + plsc.cumsum(mi) - 1
                            slc = (pos >> gsh) & nr1
                            col = pos & (gfix - 1)
                            plsc.store_scatter(dstb, [slc, col], loc, mask=m)
                            plsc.store_scatter(srcb, [slc, col],
                                               ide_v[par, sl], mask=m)
                            plsc.store_scatter(ttb, [slc, col],
                                               base + i * 16 + lane, mask=m)
                        off_vec = off_vec + cnt_vec
                    return off_vec

                off_vec = lax.fori_loop(0, ngrp // 2, grp_pair, off_vec)
                off_s = jnp.max(off_vec)
                cons, pend = pump(cons, pend, off_s)

                @pl.when(nb < nblk)
                def _():
                    pltpu.make_async_copy(
                        idr_hbm.at[pl.ds(nbase, blk_ids)], idr_v.at[npar],
                        sem3).wait()
                    pltpu.make_async_copy(
                        ide_hbm.at[pl.ds(nbase, blk_ids)], ide_v.at[npar],
                        sem4).wait()
                return (off_vec, cons, pend)

            carry0 = (jnp.zeros((16,), I32), jnp.int32(0), jnp.int32(0))
            off_vec, cons, pend = lax.fori_loop(0, nblk, blk_body, carry0)

            @pl.when(pend == 1)
            def _():
                finish_fire(((cons >> gsh) + nr1) & nr1)
            off_s = jnp.max(off_vec)
            rem = off_s - cons
            jd = (cons >> gsh) & nr1
            jd_vec = jnp.zeros((16,), I32) + jd
            for g in range(gfix // 16):
                p = g * 16 + lane
                plsc.store_scatter(dstb, [jd_vec, p], ec + lane,
                                   mask=(p >= rem))
            start_fire(jd)
            finish_fire(jd)
            plsc.subcore_barrier()
            for k in range(nzcp):
                rsl = pl.ds(row0 + k * zrows, zrows)
                pltpu.sync_copy(accum.at[rsl],
                                out_hbm.at[pl.ds(c0 + row0 + k * zrows, zrows)])
            plsc.subcore_barrier()
            return 0

        lax.fori_loop(0, npc, chunk_body, 0)

    return sc_segment


def _epilogue_body(x0_ref, seg_ref, wji_ref, bji_ref, wup_ref,
                   wb1a_ref, bb1a_ref, wb1b_ref, bb1b_ref,
                   wfbs_ref, bfbs_ref,
                   wa1a_ref, ba1a_ref, wa1b_ref, ba1b_ref,
                   wa2a_ref, ba2a_ref, wa2b_ref, ba2b_ref, out_ref):
    x0 = x0_ref[...]

    def mm(a, w):
        return _bdot(a, w[...])

    def res(x, wa, ba, wb, bb):
        h = _silu(mm(x, wa) + ba[...])
        h = _silu(mm(h, wb) + bb[...])
        return x + h

    x_ji = _silu(mm(x0, wji_ref) + bji_ref[...])
    x_kj = _silu(mm(seg_ref[...], wup_ref))
    x2 = x_ji + x_kj
    x2 = res(x2, wb1a_ref, bb1a_ref, wb1b_ref, bb1b_ref)
    x2 = _silu(mm(x2, wfbs_ref) + bfbs_ref[...])
    x = x0 + x2
    x = res(x, wa1a_ref, ba1a_ref, wa1b_ref, ba1b_ref)
    x = res(x, wa2a_ref, ba2a_ref, wa2b_ref, ba2b_ref)
    out_ref[...] = x


def _epilogue(x0, seg, w_ji, b_ji, w_up, w_bs1a, b_bs1a, w_bs1b, b_bs1b,
              w_fbs, b_fbs, w_as1a, b_as1a, w_as1b, b_as1b,
              w_as2a, b_as2a, w_as2b, b_as2b):
    blk = 2000
    grid = E // blk
    full = lambda a: pl.BlockSpec(a.shape, lambda i: (0,) * a.ndim)
    args = [w_ji, b_ji.reshape(1, EMB), w_up,
            w_bs1a, b_bs1a.reshape(1, EMB), w_bs1b, b_bs1b.reshape(1, EMB),
            w_fbs, b_fbs.reshape(1, EMB),
            w_as1a, b_as1a.reshape(1, EMB), w_as1b, b_as1b.reshape(1, EMB),
            w_as2a, b_as2a.reshape(1, EMB), w_as2b, b_as2b.reshape(1, EMB)]
    return pl.pallas_call(
        _epilogue_body,
        grid=(grid,),
        in_specs=[
            pl.BlockSpec((blk, EMB), lambda i: (i, 0)),
            pl.BlockSpec((blk, INT_EMB), lambda i: (i, 0)),
        ] + [full(a) for a in args],
        out_specs=pl.BlockSpec((blk, EMB), lambda i: (i, 0)),
        out_shape=jax.ShapeDtypeStruct((E, EMB), F32),
    )(x0, seg, *args)


def kernel(x0, rbf, sbf, id_expand_kj, id_reduce_ji, R,
           W_rbf1, W_rbf2, W_sbf1, W_sbf2, W_ji, b_ji, W_kj, b_kj,
           W_down, W_up, W_bs1a, b_bs1a, W_bs1b, b_bs1b, W_fbs, b_fbs,
           W_as1a, b_as1a, W_as1b, b_as1b, W_as2a, b_as2a, W_as2b, b_as2b):
    x_down = _prologue(x0, rbf, W_kj, b_kj, W_rbf1, W_rbf2, W_down)
    sbf_e = _sbf_transform(sbf, W_sbf1, W_sbf2)
    sc_seg = _make_sc_segment(E, T, n_chunks=16, blk_ids=1600, gfix=128)
    seg = sc_seg(x_down, sbf_e, id_reduce_ji, id_expand_kj)
    W_up = jnp.take(W_up, _seg_perm(), axis=0)
    return _epilogue(x0, seg, W_ji, b_ji, W_up, W_bs1a, b_bs1a,
                     W_bs1b, b_bs1b, W_fbs, b_fbs, W_as1a, b_as1a,
                     W_as1b, b_as1b, W_as2a, b_as2a, W_as2b, b_as2b)


# speedup vs baseline: 5.0993x; 1.1607x over previous
import functools

import jax
import jax.numpy as jnp
from jax import lax
from jax.experimental import pallas as pl
from jax.experimental.pallas import tpu as pltpu
from jax.experimental.pallas import tpu_sc as plsc

E = 320000
T = 1280000
EMB = 128
INT_EMB = 64

F32 = jnp.float32
I32 = jnp.int32


def _silu(x):
    return x * (1.0 / (1.0 + jnp.exp(-x)))


def _bdot(a, w):
    return jnp.dot(a.astype(jnp.bfloat16), w.astype(jnp.bfloat16),
                   preferred_element_type=F32)


def _prologue_body(x0_ref, rbf_ref, wkj_ref, bkj_ref, wr1_ref, wr2_ref,
                   wdn_ref, xdown_ref):
    x0 = x0_ref[...]
    h = _silu(_bdot(x0, wkj_ref[...]) + bkj_ref[...])
    rbf_e = jnp.dot(jnp.dot(rbf_ref[...], wr1_ref[...],
                            preferred_element_type=F32),
                    wr2_ref[...], preferred_element_type=F32)
    h = h * rbf_e
    xdown_ref[...] = _silu(_bdot(h, wdn_ref[...]))


def _prologue(x0, rbf, w_kj, b_kj, w_rbf1, w_rbf2, w_down):
    blk = 2000
    grid = E // blk
    full = lambda a: pl.BlockSpec(a.shape, lambda i: (0,) * a.ndim)
    b_kj2 = b_kj.reshape(1, EMB)
    return pl.pallas_call(
        _prologue_body,
        grid=(grid,),
        in_specs=[
            pl.BlockSpec((blk, EMB), lambda i: (i, 0)),
            pl.BlockSpec((blk, rbf.shape[1]), lambda i: (i, 0)),
            full(w_kj), full(b_kj2), full(w_rbf1), full(w_rbf2), full(w_down),
        ],
        out_specs=pl.BlockSpec((blk, INT_EMB), lambda i: (i, 0)),
        out_shape=jax.ShapeDtypeStruct((E, INT_EMB), F32),
    )(x0, rbf, w_kj, b_kj2, w_rbf1, w_rbf2, w_down)


def _sbf_body(sbf_ref, w1_ref, w2_ref, out_ref):
    out_ref[...] = jnp.dot(
        jnp.dot(sbf_ref[...], w1_ref[...], preferred_element_type=F32),
        w2_ref[...], preferred_element_type=F32)


def _sbf_transform(sbf, w_sbf1, w_sbf2):
    blk = 4000
    grid = T // blk
    full = lambda a: pl.BlockSpec(a.shape, lambda i: (0,) * a.ndim)
    return pl.pallas_call(
        _sbf_body,
        grid=(grid,),
        in_specs=[
            pl.BlockSpec((blk, sbf.shape[1]), lambda i: (i, 0)),
            full(w_sbf1), full(w_sbf2),
        ],
        out_specs=pl.BlockSpec((blk, INT_EMB), lambda i: (i, 0)),
        out_shape=jax.ShapeDtypeStruct((T, INT_EMB), F32),
    )(sbf, w_sbf1, w_sbf2)


def _make_sc_segment(e_total, t_total, n_chunks, blk_ids, gfix,
                     interpret=False):
    info_tiles = 16
    n_sc = 2
    ec = e_total // n_chunks
    npc = n_chunks // n_sc
    t_per_tile = t_total // info_tiles
    nblk = t_per_tile // blk_ids
    ngrp = blk_ids // 16
    d = INT_EMB
    rows_per_tile = ec // info_tiles
    zrows = next(z for z in (200, 128, 125, 100, 80, 50, 25)
                 if rows_per_tile % z == 0 and z <= gfix)
    nzcp = rows_per_tile // zrows
    nring = 16
    gsh = gfix.bit_length() - 1
    assert (1 << gsh) == gfix
    assert ec * n_chunks == e_total and t_per_tile * info_tiles == t_total
    assert nblk * blk_ids == t_per_tile and ngrp * 16 == blk_ids
    assert nzcp * zrows * info_tiles == ec and ngrp % 2 == 0
    assert nring * gfix >= blk_ids + 2 * gfix
    assert zrows <= gfix

    mesh = plsc.VectorSubcoreMesh(core_axis_name="c", subcore_axis_name="s",
                                  num_cores=n_sc, num_subcores=info_tiles)

    @functools.partial(
        pl.kernel,
        out_type=jax.ShapeDtypeStruct((e_total, d), F32),
        mesh=mesh,
        scratch_types=[
            pltpu.VMEM((2, blk_ids), I32),
            pltpu.VMEM((2, blk_ids), I32),
            pltpu.VMEM((nring, gfix), I32),
            pltpu.VMEM((nring, gfix), I32),
            pltpu.VMEM((nring, gfix), I32),
            pltpu.VMEM((gfix, d), F32),
            pltpu.VMEM((gfix, d), F32),
            pltpu.VMEM((gfix, d), F32),
            pltpu.VMEM_SHARED((ec + 16, d), F32),
            pltpu.SemaphoreType.DMA,
            pltpu.SemaphoreType.DMA,
            pltpu.SemaphoreType.DMA,
            pltpu.SemaphoreType.DMA,
        ],
        compiler_params=pltpu.CompilerParams(needs_layout_passes=False,
                                             use_tc_tiling_on_sc=False),
        interpret=interpret,
    )
    def sc_segment(xdown_hbm, sbfe_hbm, idr_hbm, ide_hbm, out_hbm,
                   idr_v, ide_v, dstb, srcb, ttb, xrows, srows, prod,
                   accum, sem1, sem2, sem3, sem4):
        sc_id = lax.axis_index("c")
        s = lax.axis_index("s")
        lane = lax.iota(I32, 16)
        nr1 = nring - 1

        for j in range(nring):
            def _init(g, _):
                v = (j * gfix + g * 16) + lane
                srcb[j, pl.ds(g * 16, 16)] = v
                ttb[j, pl.ds(g * 16, 16)] = v
                return 0
            lax.fori_loop(0, gfix // 16, _init, 0)

        def _zero_prod():
            def _zrow(r, _):
                for cg in range(d // 16):
                    prod[r, pl.ds(cg * 16, 16)] = jnp.zeros((16,), F32)
                return 0
            lax.fori_loop(0, zrows, _zrow, 0)

        def start_fire(j):
            pltpu.make_async_copy(xdown_hbm.at[srcb.at[j]], xrows, sem1).start()
            pltpu.make_async_copy(sbfe_hbm.at[ttb.at[j]], srows, sem2).start()

        def finish_fire(j):
            pltpu.make_async_copy(xdown_hbm.at[srcb.at[j]], xrows, sem1).wait()
            pltpu.make_async_copy(sbfe_hbm.at[ttb.at[j]], srows, sem2).wait()

            def _mul(r, _):
                for u in range(4):
                    row = r * 4 + u
                    for cg in range(d // 16):
                        sl = pl.ds(cg * 16, 16)
                        prod[row, sl] = xrows[row, sl] * srows[row, sl]
                return 0
            lax.fori_loop(0, gfix // 4, _mul, 0)
            pltpu.sync_copy(prod, accum.at[dstb.at[j]], add=True)

        def pump(cons, pend, off_s):
            def cond(st):
                return off_s - st[0] >= gfix

            def body(st):
                cons_, pend_ = st

                @pl.when(pend_ == 1)
                def _():
                    finish_fire(((cons_ >> gsh) + nr1) & nr1)
                start_fire((cons_ >> gsh) & nr1)
                return (cons_ + gfix, jnp.int32(1))

            return lax.while_loop(cond, body, (cons, pend))

        def chunk_body(c, _):
            c0 = (sc_id * npc + c) * ec
            row0 = s * rows_per_tile
            base0 = s * t_per_tile
            _zero_prod()
            for k in range(nzcp):
                pltpu.sync_copy(prod.at[pl.ds(0, zrows)],
                                accum.at[pl.ds(row0 + k * zrows, zrows)])
            plsc.subcore_barrier()
            pltpu.sync_copy(idr_hbm.at[pl.ds(base0, blk_ids)], idr_v.at[0])
            pltpu.sync_copy(ide_hbm.at[pl.ds(base0, blk_ids)], ide_v.at[0])

            def blk_body(b, carry):
                off_vec, cons, pend = carry
                par = b & 1
                nb = b + 1
                nbase = base0 + nb * blk_ids
                npar = nb & 1

                @pl.when(nb < nblk)
                def _():
                    pltpu.make_async_copy(
                        idr_hbm.at[pl.ds(nbase, blk_ids)], idr_v.at[npar],
                        sem3).start()
                    pltpu.make_async_copy(
                        ide_hbm.at[pl.ds(nbase, blk_ids)], ide_v.at[npar],
                        sem4).start()

                base = base0 + b * blk_ids

                def grp_pair(ip, off_vec):
                    for u in range(2):
                        i = ip * 2 + u
                        sl = pl.ds(i * 16, 16)
                        loc = idr_v[par, sl] - c0
                        m = loc.astype(jnp.uint32) < jnp.uint32(ec)
                        cnt_vec = plsc.all_reduce_population_count(m)

                        @pl.when(jnp.any(m))
                        def _(loc=loc, m=m, i=i, off_vec=off_vec, sl=sl):
                            mi = m.astype(I32)
                            pos = off_vec + plsc.cumsum(mi) - 1
                            slc = (pos >> gsh) & nr1
                            col = pos & (gfix - 1)
                            plsc.store_scatter(dstb, [slc, col], loc, mask=m)
                            plsc.store_scatter(srcb, [slc, col],
                                               ide_v[par, sl], mask=m)
                            plsc.store_scatter(ttb, [slc, col],
                                               base + i * 16 + lane, mask=m)
                        off_vec = off_vec + cnt_vec
                    return off_vec

                off_vec = lax.fori_loop(0, ngrp // 2, grp_pair, off_vec)
                off_s = jnp.max(off_vec)
                cons, pend = pump(cons, pend, off_s)

                @pl.when(nb < nblk)
                def _():
                    pltpu.make_async_copy(
                        idr_hbm.at[pl.ds(nbase, blk_ids)], idr_v.at[npar],
                        sem3).wait()
                    pltpu.make_async_copy(
                        ide_hbm.at[pl.ds(nbase, blk_ids)], ide_v.at[npar],
                        sem4).wait()
                return (off_vec, cons, pend)

            carry0 = (jnp.zeros((16,), I32), jnp.int32(0), jnp.int32(0))
            off_vec, cons, pend = lax.fori_loop(0, nblk, blk_body, carry0)

            @pl.when(pend == 1)
            def _():
                finish_fire(((cons >> gsh) + nr1) & nr1)
            off_s = jnp.max(off_vec)
            rem = off_s - cons
            jd = (cons >> gsh) & nr1
            jd_vec = jnp.zeros((16,), I32) + jd
            for g in range(gfix // 16):
                p = g * 16 + lane
                plsc.store_scatter(dstb, [jd_vec, p], ec + lane,
                                   mask=(p >= rem))
            start_fire(jd)
            finish_fire(jd)
            plsc.subcore_barrier()
            for k in range(nzcp):
                rsl = pl.ds(row0 + k * zrows, zrows)
                pltpu.sync_copy(accum.at[rsl],
                                out_hbm.at[pl.ds(c0 + row0 + k * zrows, zrows)])
            plsc.subcore_barrier()
            return 0

        lax.fori_loop(0, npc, chunk_body, 0)

    return sc_segment


def _epilogue_body(x0_ref, seg_ref, wji_ref, bji_ref, wup_ref,
                   wb1a_ref, bb1a_ref, wb1b_ref, bb1b_ref,
                   wfbs_ref, bfbs_ref,
                   wa1a_ref, ba1a_ref, wa1b_ref, ba1b_ref,
                   wa2a_ref, ba2a_ref, wa2b_ref, ba2b_ref, out_ref):
    x0 = x0_ref[...]

    def mm(a, w):
        return _bdot(a, w[...])

    def res(x, wa, ba, wb, bb):
        h = _silu(mm(x, wa) + ba[...])
        h = _silu(mm(h, wb) + bb[...])
        return x + h

    x_ji = _silu(mm(x0, wji_ref) + bji_ref[...])
    x_kj = _silu(mm(seg_ref[...], wup_ref))
    x2 = x_ji + x_kj
    x2 = res(x2, wb1a_ref, bb1a_ref, wb1b_ref, bb1b_ref)
    x2 = _silu(mm(x2, wfbs_ref) + bfbs_ref[...])
    x = x0 + x2
    x = res(x, wa1a_ref, ba1a_ref, wa1b_ref, ba1b_ref)
    x = res(x, wa2a_ref, ba2a_ref, wa2b_ref, ba2b_ref)
    out_ref[...] = x


def _epilogue(x0, seg, w_ji, b_ji, w_up, w_bs1a, b_bs1a, w_bs1b, b_bs1b,
              w_fbs, b_fbs, w_as1a, b_as1a, w_as1b, b_as1b,
              w_as2a, b_as2a, w_as2b, b_as2b):
    blk = 2000
    grid = E // blk
    full = lambda a: pl.BlockSpec(a.shape, lambda i: (0,) * a.ndim)
    args = [w_ji, b_ji.reshape(1, EMB), w_up,
            w_bs1a, b_bs1a.reshape(1, EMB), w_bs1b, b_bs1b.reshape(1, EMB),
            w_fbs, b_fbs.reshape(1, EMB),
            w_as1a, b_as1a.reshape(1, EMB), w_as1b, b_as1b.reshape(1, EMB),
            w_as2a, b_as2a.reshape(1, EMB), w_as2b, b_as2b.reshape(1, EMB)]
    return pl.pallas_call(
        _epilogue_body,
        grid=(grid,),
        in_specs=[
            pl.BlockSpec((blk, EMB), lambda i: (i, 0)),
            pl.BlockSpec((blk, INT_EMB), lambda i: (i, 0)),
        ] + [full(a) for a in args],
        out_specs=pl.BlockSpec((blk, EMB), lambda i: (i, 0)),
        out_shape=jax.ShapeDtypeStruct((E, EMB), F32),
    )(x0, seg, *args)


def kernel(x0, rbf, sbf, id_expand_kj, id_reduce_ji, R,
           W_rbf1, W_rbf2, W_sbf1, W_sbf2, W_ji, b_ji, W_kj, b_kj,
           W_down, W_up, W_bs1a, b_bs1a, W_bs1b, b_bs1b, W_fbs, b_fbs,
           W_as1a, b_as1a, W_as1b, b_as1b, W_as2a, b_as2a, W_as2b, b_as2b):
    x_down = _prologue(x0, rbf, W_kj, b_kj, W_rbf1, W_rbf2, W_down)
    sbf_e = _sbf_transform(sbf, W_sbf1, W_sbf2)
    sc_seg = _make_sc_segment(E, T, n_chunks=16, blk_ids=1600, gfix=128)
    seg = sc_seg(x_down, sbf_e, id_reduce_ji, id_expand_kj)
    return _epilogue(x0, seg, W_ji, b_ji, W_up, W_bs1a, b_bs1a,
                     W_bs1b, b_bs1b, W_fbs, b_fbs, W_as1a, b_as1a,
                     W_as1b, b_as1b, W_as2a, b_as2a, W_as2b, b_as2b)
